# baseline (device time: 61043 ns/iter reference)
import jax
import jax.numpy as jnp
from jax import lax
from jax.experimental import pallas as pl
from jax.experimental.pallas import tpu as pltpu


def kernel(x, w_mat):
    m, _ = x.shape
    _, n = w_mat.shape
    h2, h4, h8 = m // 2, m // 4, m // 8

    def body(x_ref, w_ref, out_ref, acc_ref, g_ref, r0_ref, r1_ref, r2_ref,
             send_sems, recv_sems):
        i = lax.axis_index("i")
        q = i & 3
        ycrd = q >> 1
        xcrd = (q & 1) ^ ycrd
        zcrd = i >> 2

        acc_ref[...] = jnp.dot(
            x_ref[...], w_ref[...], preferred_element_type=jnp.float32
        ).astype(jnp.bfloat16)

        def exchange(r, partner, src, dst):
            rdma = pltpu.make_async_remote_copy(
                src_ref=src,
                dst_ref=dst,
                send_sem=send_sems.at[r],
                recv_sem=recv_sems.at[r],
                device_id=(partner,),
                device_id_type=pl.DeviceIdType.MESH,
            )
            rdma.start()
            rdma.wait()

        k0 = ycrd * h2
        exchange(0, i ^ 3, acc_ref.at[pl.ds((1 - ycrd) * h2, h2)], r0_ref)
        acc_ref[pl.ds(k0, h2), :] = acc_ref[pl.ds(k0, h2), :] + r0_ref[...]

        k1 = k0 + xcrd * h4
        exchange(1, i ^ 1, acc_ref.at[pl.ds(k0 + (1 - xcrd) * h4, h4)], r1_ref)
        acc_ref[pl.ds(k1, h4), :] = acc_ref[pl.ds(k1, h4), :] + r1_ref[...]

        k2 = k1 + zcrd * h8
        exchange(2, i ^ 4, acc_ref.at[pl.ds(k1 + (1 - zcrd) * h8, h8)], r2_ref)

        red = acc_ref[pl.ds(k2, h8), :] + r2_ref[...]
        g_ref[pl.ds(k2, h8), :] = jnp.maximum(red, 0)

        exchange(3, i ^ 4, g_ref.at[pl.ds(k2, h8)], g_ref.at[pl.ds(k2, h8)])
        exchange(4, i ^ 1, g_ref.at[pl.ds(k1, h4)], g_ref.at[pl.ds(k1, h4)])
        exchange(5, i ^ 3, g_ref.at[pl.ds(k0, h2)], g_ref.at[pl.ds(k0, h2)])

        out_ref[...] = g_ref[...].astype(jnp.float32)

    return pl.pallas_call(
        body,
        out_shape=jax.ShapeDtypeStruct((m, n), jnp.float32),
        in_specs=[
            pl.BlockSpec(memory_space=pltpu.VMEM),
            pl.BlockSpec(memory_space=pltpu.VMEM),
        ],
        out_specs=pl.BlockSpec(memory_space=pltpu.VMEM),
        scratch_shapes=[
            pltpu.VMEM((m, n), jnp.bfloat16),
            pltpu.VMEM((m, n), jnp.bfloat16),
            pltpu.VMEM((h2, n), jnp.bfloat16),
            pltpu.VMEM((h4, n), jnp.bfloat16),
            pltpu.VMEM((h8, n), jnp.bfloat16),
            pltpu.SemaphoreType.DMA((6,)),
            pltpu.SemaphoreType.DMA((6,)),
        ],
    )(x, w_mat)


# device time: 42705 ns/iter; 1.4294x vs baseline; 1.4294x over previous
import jax
import jax.numpy as jnp
from jax import lax
from jax.experimental import pallas as pl
from jax.experimental.pallas import tpu as pltpu


def kernel(x, w_mat):
    m, _ = x.shape
    _, n = w_mat.shape
    h2, h4, h8 = m // 2, m // 4, m // 8
    nh = n // 2

    def body(x_ref, w_ref, out_ref, acc_ref, g_ref,
             ra0, ra1, ra2, rb0, rb1, rb2, send_sems, recv_sems):
        i = lax.axis_index("i")
        q = i & 3
        ycrd = q >> 1
        xcrd = (q & 1) ^ ycrd
        zcrd = i >> 2
        py, px, pz = i ^ 3, i ^ 1, i ^ 4

        acc_ref[...] = jnp.dot(
            x_ref[...], w_ref[...], preferred_element_type=jnp.float32
        ).astype(jnp.bfloat16)

        def start(r, partner, src, dst):
            rdma = pltpu.make_async_remote_copy(
                src_ref=src,
                dst_ref=dst,
                send_sem=send_sems.at[r],
                recv_sem=recv_sems.at[r],
                device_id=(partner,),
                device_id_type=pl.DeviceIdType.MESH,
            )
            rdma.start()
            return rdma

        A = pl.ds(0, nh)
        B = pl.ds(nh, nh)

        kA0 = ycrd * h2
        kB0 = xcrd * h2
        a = start(0, py, acc_ref.at[pl.ds((1 - ycrd) * h2, h2), A], ra0)
        b = start(1, px, acc_ref.at[pl.ds((1 - xcrd) * h2, h2), B], rb0)
        a.wait()
        acc_ref[pl.ds(kA0, h2), A] = acc_ref[pl.ds(kA0, h2), A] + ra0[...]
        b.wait()
        acc_ref[pl.ds(kB0, h2), B] = acc_ref[pl.ds(kB0, h2), B] + rb0[...]

        kA1 = kA0 + xcrd * h4
        kB1 = kB0 + zcrd * h4
        a = start(2, px, acc_ref.at[pl.ds(kA0 + (1 - xcrd) * h4, h4), A], ra1)
        b = start(3, pz, acc_ref.at[pl.ds(kB0 + (1 - zcrd) * h4, h4), B], rb1)
        a.wait()
        acc_ref[pl.ds(kA1, h4), A] = acc_ref[pl.ds(kA1, h4), A] + ra1[...]
        b.wait()
        acc_ref[pl.ds(kB1, h4), B] = acc_ref[pl.ds(kB1, h4), B] + rb1[...]

        kA2 = kA1 + zcrd * h8
        kB2 = kB1 + ycrd * h8
        a = start(4, pz, acc_ref.at[pl.ds(kA1 + (1 - zcrd) * h8, h8), A], ra2)
        b = start(5, py, acc_ref.at[pl.ds(kB1 + (1 - ycrd) * h8, h8), B], rb2)

        a.wait()
        g_ref[pl.ds(kA2, h8), A] = jnp.maximum(
            acc_ref[pl.ds(kA2, h8), A] + ra2[...], 0
        )
        b.wait()
        g_ref[pl.ds(kB2, h8), B] = jnp.maximum(
            acc_ref[pl.ds(kB2, h8), B] + rb2[...], 0
        )

        a = start(6, pz, g_ref.at[pl.ds(kA2, h8), A], g_ref.at[pl.ds(kA2, h8), A])
        b = start(7, py, g_ref.at[pl.ds(kB2, h8), B], g_ref.at[pl.ds(kB2, h8), B])
        a.wait()
        b.wait()
        a = start(8, px, g_ref.at[pl.ds(kA1, h4), A], g_ref.at[pl.ds(kA1, h4), A])
        b = start(9, pz, g_ref.at[pl.ds(kB1, h4), B], g_ref.at[pl.ds(kB1, h4), B])
        a.wait()
        b.wait()
        a = start(10, py, g_ref.at[pl.ds(kA0, h2), A], g_ref.at[pl.ds(kA0, h2), A])
        b = start(11, px, g_ref.at[pl.ds(kB0, h2), B], g_ref.at[pl.ds(kB0, h2), B])
        a.wait()
        b.wait()

        out_ref[...] = g_ref[...].astype(jnp.float32)

    return pl.pallas_call(
        body,
        out_shape=jax.ShapeDtypeStruct((m, n), jnp.float32),
        in_specs=[
            pl.BlockSpec(memory_space=pltpu.VMEM),
            pl.BlockSpec(memory_space=pltpu.VMEM),
        ],
        out_specs=pl.BlockSpec(memory_space=pltpu.VMEM),
        scratch_shapes=[
            pltpu.VMEM((m, n), jnp.bfloat16),
            pltpu.VMEM((m, n), jnp.bfloat16),
            pltpu.VMEM((h2, nh), jnp.bfloat16),
            pltpu.VMEM((h4, nh), jnp.bfloat16),
            pltpu.VMEM((h8, nh), jnp.bfloat16),
            pltpu.VMEM((h2, nh), jnp.bfloat16),
            pltpu.VMEM((h4, nh), jnp.bfloat16),
            pltpu.VMEM((h8, nh), jnp.bfloat16),
            pltpu.SemaphoreType.DMA((12,)),
            pltpu.SemaphoreType.DMA((12,)),
        ],
    )(x, w_mat)


# device time: 33005 ns/iter; 1.8495x vs baseline; 1.2939x over previous
import jax
import jax.numpy as jnp
from jax import lax
from jax.experimental import pallas as pl
from jax.experimental.pallas import tpu as pltpu


def kernel(x, w_mat):
    m, _ = x.shape
    _, n = w_mat.shape
    h2, h4, h8 = m // 2, m // 4, m // 8
    col_slices = [(0, 3 * n // 8), (3 * n // 8, 3 * n // 8), (6 * n // 8, n // 4)]

    def body(x_ref, w_ref, out_ref, acc_ref, g_ref,
             r0a, r0b, r0c, r1a, r1b, r1c, r2a, r2b, r2c,
             send_sems, recv_sems):
        i = lax.axis_index("i")
        q = i & 3
        ycrd = q >> 1
        xcrd = (q & 1) ^ ycrd
        zcrd = i >> 2
        py, px, pz = i ^ 3, i ^ 1, i ^ 4

        orders = [
            ((ycrd, xcrd, zcrd), (py, px, pz)),
            ((xcrd, zcrd, ycrd), (px, pz, py)),
            ((zcrd, ycrd, xcrd), (pz, py, px)),
        ]
        cols = [pl.ds(off, w) for off, w in col_slices]
        r0refs = [r0a, r0b, r0c]
        r1refs = [r1a, r1b, r1c]
        r2refs = [r2a, r2b, r2c]

        k0 = [c[0] * h2 for c, _ in orders]
        k1 = [k0[t] + orders[t][0][1] * h4 for t in range(3)]
        k2 = [k1[t] + orders[t][0][2] * h8 for t in range(3)]
        s0 = [(1 - orders[t][0][0]) * h2 for t in range(3)]
        s1 = [k0[t] + (1 - orders[t][0][1]) * h4 for t in range(3)]
        s2 = [k1[t] + (1 - orders[t][0][2]) * h8 for t in range(3)]
        sib0, sib1, sib2 = s2, s1, s0

        barrier_sem = pltpu.get_barrier_semaphore()
        for nbr in (py, px, pz):
            pl.semaphore_signal(
                barrier_sem, inc=1,
                device_id=(nbr,), device_id_type=pl.DeviceIdType.MESH,
            )

        def mm(rows, t):
            return jnp.dot(
                x_ref[rows, :], w_ref[:, cols[t]],
                preferred_element_type=jnp.float32,
            ).astype(jnp.bfloat16)

        for t in range(3):
            acc_ref[pl.ds(s0[t], h2), cols[t]] = mm(pl.ds(s0[t], h2), t)

        pl.semaphore_wait(barrier_sem, 3)

        def start(r, partner, src, dst):
            rdma = pltpu.make_async_remote_copy(
                src_ref=src, dst_ref=dst,
                send_sem=send_sems.at[r], recv_sem=recv_sems.at[r],
                device_id=(partner,), device_id_type=pl.DeviceIdType.MESH,
            )
            rdma.start()
            return rdma

        rd = [start(t, orders[t][1][0],
                    acc_ref.at[pl.ds(s0[t], h2), cols[t]], r0refs[t])
              for t in range(3)]
        for t in range(3):
            acc_ref[pl.ds(k0[t], h2), cols[t]] = mm(pl.ds(k0[t], h2), t)
        for t in range(3):
            rd[t].wait()
            acc_ref[pl.ds(k0[t], h2), cols[t]] = (
                acc_ref[pl.ds(k0[t], h2), cols[t]] + r0refs[t][...]
            )

        rd = [start(3 + t, orders[t][1][1],
                    acc_ref.at[pl.ds(s1[t], h4), cols[t]], r1refs[t])
              for t in range(3)]
        for t in range(3):
            rd[t].wait()
            acc_ref[pl.ds(k1[t], h4), cols[t]] = (
                acc_ref[pl.ds(k1[t], h4), cols[t]] + r1refs[t][...]
            )

        rd = [start(6 + t, orders[t][1][2],
                    acc_ref.at[pl.ds(s2[t], h8), cols[t]], r2refs[t])
              for t in range(3)]
        for t in range(3):
            rd[t].wait()
            g_ref[pl.ds(k2[t], h8), cols[t]] = jnp.maximum(
                acc_ref[pl.ds(k2[t], h8), cols[t]] + r2refs[t][...], 0
            )

        rd = [start(9 + t, orders[t][1][2],
                    g_ref.at[pl.ds(k2[t], h8), cols[t]],
                    g_ref.at[pl.ds(k2[t], h8), cols[t]]) for t in range(3)]
        for t in range(3):
            out_ref[pl.ds(k2[t], h8), cols[t]] = (
                g_ref[pl.ds(k2[t], h8), cols[t]].astype(jnp.float32)
            )
        for t in range(3):
            rd[t].wait()

        rd = [start(12 + t, orders[t][1][1],
                    g_ref.at[pl.ds(k1[t], h4), cols[t]],
                    g_ref.at[pl.ds(k1[t], h4), cols[t]]) for t in range(3)]
        for t in range(3):
            out_ref[pl.ds(sib0[t], h8), cols[t]] = (
                g_ref[pl.ds(sib0[t], h8), cols[t]].astype(jnp.float32)
            )
        for t in range(3):
            rd[t].wait()

        rd = [start(15 + t, orders[t][1][0],
                    g_ref.at[pl.ds(k0[t], h2), cols[t]],
                    g_ref.at[pl.ds(k0[t], h2), cols[t]]) for t in range(3)]
        for t in range(3):
            out_ref[pl.ds(sib1[t], h4), cols[t]] = (
                g_ref[pl.ds(sib1[t], h4), cols[t]].astype(jnp.float32)
            )
        for t in range(3):
            rd[t].wait()
            out_ref[pl.ds(sib2[t], h2), cols[t]] = (
                g_ref[pl.ds(sib2[t], h2), cols[t]].astype(jnp.float32)
            )

    return pl.pallas_call(
        body,
        out_shape=jax.ShapeDtypeStruct((m, n), jnp.float32),
        in_specs=[
            pl.BlockSpec(memory_space=pltpu.VMEM),
            pl.BlockSpec(memory_space=pltpu.VMEM),
        ],
        out_specs=pl.BlockSpec(memory_space=pltpu.VMEM),
        scratch_shapes=[
            pltpu.VMEM((m, n), jnp.bfloat16),
            pltpu.VMEM((m, n), jnp.bfloat16),
            pltpu.VMEM((h2, col_slices[0][1]), jnp.bfloat16),
            pltpu.VMEM((h2, col_slices[1][1]), jnp.bfloat16),
            pltpu.VMEM((h2, col_slices[2][1]), jnp.bfloat16),
            pltpu.VMEM((h4, col_slices[0][1]), jnp.bfloat16),
            pltpu.VMEM((h4, col_slices[1][1]), jnp.bfloat16),
            pltpu.VMEM((h4, col_slices[2][1]), jnp.bfloat16),
            pltpu.VMEM((h8, col_slices[0][1]), jnp.bfloat16),
            pltpu.VMEM((h8, col_slices[1][1]), jnp.bfloat16),
            pltpu.VMEM((h8, col_slices[2][1]), jnp.bfloat16),
            pltpu.SemaphoreType.DMA((18,)),
            pltpu.SemaphoreType.DMA((18,)),
        ],
        compiler_params=pltpu.CompilerParams(collective_id=0),
    )(x, w_mat)


# device time: 30215 ns/iter; 2.0203x vs baseline; 1.0923x over previous
import jax
import jax.numpy as jnp
from jax import lax
from jax.experimental import pallas as pl
from jax.experimental.pallas import tpu as pltpu

_GROUPS = [(0, 256), (256, 256), (512, 128), (640, 128), (768, 128), (896, 128)]
_NG = len(_GROUPS)


def kernel(x, w_mat):
    m, _ = x.shape
    _, n = w_mat.shape
    h2, h4, h8 = m // 2, m // 4, m // 8

    def body(x_ref, w_ref, out_ref, acc_ref, g_ref, *rest):
        recvs = [rest[3 * g: 3 * g + 3] for g in range(_NG)]
        send_sems, recv_sems = rest[3 * _NG], rest[3 * _NG + 1]

        i = lax.axis_index("i")
        q = i & 3
        ycrd = q >> 1
        xcrd = (q & 1) ^ ycrd
        zcrd = i >> 2
        py, px, pz = i ^ 3, i ^ 1, i ^ 4

        orders3 = [
            ((ycrd, xcrd, zcrd), (py, px, pz)),
            ((xcrd, zcrd, ycrd), (px, pz, py)),
            ((zcrd, ycrd, xcrd), (pz, py, px)),
        ]
        crd = [orders3[g % 3][0] for g in range(_NG)]
        par = [orders3[g % 3][1] for g in range(_NG)]
        cols = [pl.ds(off, w) for off, w in _GROUPS]

        k0 = [crd[g][0] * h2 for g in range(_NG)]
        k1 = [k0[g] + crd[g][1] * h4 for g in range(_NG)]
        k2 = [k1[g] + crd[g][2] * h8 for g in range(_NG)]
        s0 = [(1 - crd[g][0]) * h2 for g in range(_NG)]
        s1 = [k0[g] + (1 - crd[g][1]) * h4 for g in range(_NG)]
        s2 = [k1[g] + (1 - crd[g][2]) * h8 for g in range(_NG)]
        sib0, sib1, sib2 = s2, s1, s0

        barrier_sem = pltpu.get_barrier_semaphore()
        for nbr in (py, px, pz):
            pl.semaphore_signal(
                barrier_sem, inc=1,
                device_id=(nbr,), device_id_type=pl.DeviceIdType.MESH,
            )

        def mm(rows, g):
            return jnp.dot(
                x_ref[rows, :], w_ref[:, cols[g]],
                preferred_element_type=jnp.float32,
            ).astype(jnp.bfloat16)

        for g in range(_NG):
            acc_ref[pl.ds(s0[g], h2), cols[g]] = mm(pl.ds(s0[g], h2), g)

        pl.semaphore_wait(barrier_sem, 3)

        def start(r, g, partner, src, dst):
            rdma = pltpu.make_async_remote_copy(
                src_ref=src, dst_ref=dst,
                send_sem=send_sems.at[6 * g + r], recv_sem=recv_sems.at[6 * g + r],
                device_id=(partner,), device_id_type=pl.DeviceIdType.MESH,
            )
            rdma.start()
            return rdma

        def start_rs(r, g, soff, rows):
            return start(r, g, par[g][r],
                         acc_ref.at[pl.ds(soff, rows), cols[g]], recvs[g][r])

        def start_ag(r, g, koff, rows):
            blk = g_ref.at[pl.ds(koff, rows), cols[g]]
            return start(3 + r, g, par[g][2 - r], blk, blk)

        rd = [start_rs(0, g, s0[g], h2) for g in range(_NG)]
        for g in range(_NG):
            acc_ref[pl.ds(k0[g], h2), cols[g]] = mm(pl.ds(k0[g], h2), g)

        rd1 = []
        for g in range(_NG):
            rd[g].wait()
            acc_ref[pl.ds(k0[g], h2), cols[g]] = (
                acc_ref[pl.ds(k0[g], h2), cols[g]] + recvs[g][0][...]
            )
            rd1.append(start_rs(1, g, s1[g], h4))

        rd2 = []
        for g in range(_NG):
            rd1[g].wait()
            acc_ref[pl.ds(k1[g], h4), cols[g]] = (
                acc_ref[pl.ds(k1[g], h4), cols[g]] + recvs[g][1][...]
            )
            rd2.append(start_rs(2, g, s2[g], h8))

        ag0 = []
        for g in range(_NG):
            rd2[g].wait()
            g_ref[pl.ds(k2[g], h8), cols[g]] = jnp.maximum(
                acc_ref[pl.ds(k2[g], h8), cols[g]] + recvs[g][2][...], 0
            )
            ag0.append(start_ag(0, g, k2[g], h8))

        for g in range(_NG):
            out_ref[pl.ds(k2[g], h8), cols[g]] = (
                g_ref[pl.ds(k2[g], h8), cols[g]].astype(jnp.float32)
            )

        ag1 = []
        for g in range(_NG):
            ag0[g].wait()
            ag1.append(start_ag(1, g, k1[g], h4))
        for g in range(_NG):
            out_ref[pl.ds(sib0[g], h8), cols[g]] = (
                g_ref[pl.ds(sib0[g], h8), cols[g]].astype(jnp.float32)
            )

        ag2 = []
        for g in range(_NG):
            ag1[g].wait()
            ag2.append(start_ag(2, g, k0[g], h2))
        for g in range(_NG):
            out_ref[pl.ds(sib1[g], h4), cols[g]] = (
                g_ref[pl.ds(sib1[g], h4), cols[g]].astype(jnp.float32)
            )

        for g in range(_NG):
            ag2[g].wait()
            out_ref[pl.ds(sib2[g], h2), cols[g]] = (
                g_ref[pl.ds(sib2[g], h2), cols[g]].astype(jnp.float32)
            )

    recv_shapes = []
    for _, w in _GROUPS:
        recv_shapes += [
            pltpu.VMEM((h2, w), jnp.bfloat16),
            pltpu.VMEM((h4, w), jnp.bfloat16),
            pltpu.VMEM((h8, w), jnp.bfloat16),
        ]

    return pl.pallas_call(
        body,
        out_shape=jax.ShapeDtypeStruct((m, n), jnp.float32),
        in_specs=[
            pl.BlockSpec(memory_space=pltpu.VMEM),
            pl.BlockSpec(memory_space=pltpu.VMEM),
        ],
        out_specs=pl.BlockSpec(memory_space=pltpu.VMEM),
        scratch_shapes=[
            pltpu.VMEM((m, n), jnp.bfloat16),
            pltpu.VMEM((m, n), jnp.bfloat16),
            *recv_shapes,
            pltpu.SemaphoreType.DMA((6 * _NG,)),
            pltpu.SemaphoreType.DMA((6 * _NG,)),
        ],
        compiler_params=pltpu.CompilerParams(collective_id=0),
    )(x, w_mat)


# device time: 28117 ns/iter; 2.1710x vs baseline; 1.0746x over previous
import jax
import jax.numpy as jnp
from jax import lax
from jax.experimental import pallas as pl
from jax.experimental.pallas import tpu as pltpu

_GROUPS = [(0, 256), (256, 256), (512, 128), (640, 128), (768, 128), (896, 128)]
_NG = len(_GROUPS)
_COMM = True


def kernel(x, w_mat):
    m, _ = x.shape
    _, n = w_mat.shape
    h2, h4 = m // 2, m // 4

    def body(x_ref, w_ref, out_ref, acc_ref, *rest):
        recvs = [rest[3 * g: 3 * g + 3] for g in range(_NG)]
        send_sems, recv_sems = rest[3 * _NG], rest[3 * _NG + 1]

        i = lax.axis_index("i")
        q = i & 3
        ycrd = q >> 1
        xcrd = (q & 1) ^ ycrd
        zcrd = i >> 2
        py, px, pz = i ^ 3, i ^ 1, i ^ 4

        orders3 = [
            ((ycrd, xcrd, zcrd), (py, px, pz)),
            ((xcrd, zcrd, ycrd), (px, pz, py)),
            ((zcrd, ycrd, xcrd), (pz, py, px)),
        ]
        crd = [orders3[g % 3][0] for g in range(_NG)]
        par = [orders3[g % 3][1] for g in range(_NG)]
        cols = [pl.ds(off, w) for off, w in _GROUPS]

        k0 = [crd[g][0] * h2 for g in range(_NG)]
        k1 = [k0[g] + crd[g][1] * h4 for g in range(_NG)]
        s0 = [(1 - crd[g][0]) * h2 for g in range(_NG)]
        s1 = [k0[g] + (1 - crd[g][1]) * h4 for g in range(_NG)]

        if _COMM:
            barrier_sem = pltpu.get_barrier_semaphore()
            for nbr in (py, px, pz):
                pl.semaphore_signal(
                    barrier_sem, inc=1,
                    device_id=(nbr,), device_id_type=pl.DeviceIdType.MESH,
                )

        def mm(rows, g):
            return jnp.dot(
                x_ref[rows, :], w_ref[:, cols[g]],
                preferred_element_type=jnp.float32,
            ).astype(jnp.bfloat16)

        for g in range(_NG):
            acc_ref[pl.ds(s0[g], h2), cols[g]] = mm(pl.ds(s0[g], h2), g)

        if _COMM:
            pl.semaphore_wait(barrier_sem, 3)

        class _Noop:
            def wait(self):
                pass

        def start(r, g, partner, src, dst):
            if not _COMM:
                return _Noop()
            rdma = pltpu.make_async_remote_copy(
                src_ref=src, dst_ref=dst,
                send_sem=send_sems.at[5 * g + r], recv_sem=recv_sems.at[5 * g + r],
                device_id=(partner,), device_id_type=pl.DeviceIdType.MESH,
            )
            rdma.start()
            return rdma

        rd = [start(0, g, par[g][0],
                    acc_ref.at[pl.ds(s0[g], h2), cols[g]], recvs[g][0])
              for g in range(_NG)]
        for g in range(_NG):
            acc_ref[pl.ds(k0[g], h2), cols[g]] = mm(pl.ds(k0[g], h2), g)

        rd1 = []
        for g in range(_NG):
            rd[g].wait()
            acc_ref[pl.ds(k0[g], h2), cols[g]] = (
                acc_ref[pl.ds(k0[g], h2), cols[g]] + recvs[g][0][...]
            )
            rd1.append(start(1, g, par[g][1],
                             acc_ref.at[pl.ds(s1[g], h4), cols[g]], recvs[g][1]))

        rd2 = []
        for g in range(_NG):
            rd1[g].wait()
            acc_ref[pl.ds(k1[g], h4), cols[g]] = (
                acc_ref[pl.ds(k1[g], h4), cols[g]] + recvs[g][1][...]
            )
            rd2.append(start(2, g, par[g][2],
                             acc_ref.at[pl.ds(k1[g], h4), cols[g]], recvs[g][2]))

        ag1 = []
        for g in range(_NG):
            rd2[g].wait()
            acc_ref[pl.ds(k1[g], h4), cols[g]] = jnp.maximum(
                acc_ref[pl.ds(k1[g], h4), cols[g]] + recvs[g][2][...], 0
            )
            blk = acc_ref.at[pl.ds(k1[g], h4), cols[g]]
            ag1.append(start(3, g, par[g][1], blk, blk))

        for g in range(_NG):
            out_ref[pl.ds(k1[g], h4), cols[g]] = (
                acc_ref[pl.ds(k1[g], h4), cols[g]].astype(jnp.float32)
            )

        ag2 = []
        for g in range(_NG):
            ag1[g].wait()
            blk = acc_ref.at[pl.ds(k0[g], h2), cols[g]]
            ag2.append(start(4, g, par[g][0], blk, blk))
        for g in range(_NG):
            out_ref[pl.ds(s1[g], h4), cols[g]] = (
                acc_ref[pl.ds(s1[g], h4), cols[g]].astype(jnp.float32)
            )

        for g in range(_NG):
            ag2[g].wait()
            out_ref[pl.ds(s0[g], h2), cols[g]] = (
                acc_ref[pl.ds(s0[g], h2), cols[g]].astype(jnp.float32)
            )

    recv_shapes = []
    for _, w in _GROUPS:
        recv_shapes += [
            pltpu.VMEM((h2, w), jnp.bfloat16),
            pltpu.VMEM((h4, w), jnp.bfloat16),
            pltpu.VMEM((h4, w), jnp.bfloat16),
        ]

    return pl.pallas_call(
        body,
        out_shape=jax.ShapeDtypeStruct((m, n), jnp.float32),
        in_specs=[
            pl.BlockSpec(memory_space=pltpu.VMEM),
            pl.BlockSpec(memory_space=pltpu.VMEM),
        ],
        out_specs=pl.BlockSpec(memory_space=pltpu.VMEM),
        scratch_shapes=[
            pltpu.VMEM((m, n), jnp.bfloat16),
            *recv_shapes,
            pltpu.SemaphoreType.DMA((5 * _NG,)),
            pltpu.SemaphoreType.DMA((5 * _NG,)),
        ],
        compiler_params=(
            pltpu.CompilerParams(collective_id=0) if _COMM
            else pltpu.CompilerParams()
        ),
    )(x, w_mat)
